# baseline (speedup 1.0000x reference)
import jax
import jax.numpy as jnp
from jax.experimental import pallas as pl
from jax.experimental.pallas import tpu as pltpu

_BATCH = 1


def _scale_kernel(stage_ref, emb_t_ref, x_ref, o_ref):
    s = stage_ref[0]
    emb_t = emb_t_ref[...]
    col = jax.lax.broadcasted_iota(jnp.int32, emb_t.shape, 1)
    scale = jnp.sum(jnp.where(col == s, emb_t, 0.0), axis=1)
    o_ref[...] = x_ref[...] * scale[None, :, None]


def kernel(x, stage, embeddings):
    b, c, h, w = x.shape
    x3 = x.reshape(b, c, h * w)
    stage_arr = jnp.asarray(stage, jnp.int32).reshape((1,))
    emb_t = embeddings.T

    out = pl.pallas_call(
        _scale_kernel,
        grid=(b // _BATCH,),
        in_specs=[
            pl.BlockSpec(memory_space=pltpu.SMEM),
            pl.BlockSpec((c, embeddings.shape[0]), lambda i: (0, 0)),
            pl.BlockSpec((_BATCH, c, h * w), lambda i: (i, 0, 0)),
        ],
        out_specs=pl.BlockSpec((_BATCH, c, h * w), lambda i: (i, 0, 0)),
        out_shape=jax.ShapeDtypeStruct((b, c, h * w), x.dtype),
        compiler_params=pltpu.CompilerParams(
            dimension_semantics=("parallel",),
        ),
    )(stage_arr, emb_t, x3)
    return out.reshape(b, c, h, w)


# speedup vs baseline: 1.0583x; 1.0583x over previous
import jax
import jax.numpy as jnp
from jax.experimental import pallas as pl
from jax.experimental.pallas import tpu as pltpu

_K = 8


def _scale_body(stage_ref, emb_t_ref, x_hbm, o_hbm, in_buf, out_buf,
                in_sem, out_sem):
    n = x_hbm.shape[0]
    s = stage_ref[0]
    emb_t = emb_t_ref[...]
    col = jax.lax.broadcasted_iota(jnp.int32, emb_t.shape, 1)
    scale = jnp.sum(jnp.where(col == s, emb_t, 0.0), axis=1)[:, None]

    def in_copy(i, slot):
        return pltpu.make_async_copy(x_hbm.at[i], in_buf.at[slot],
                                     in_sem.at[slot])

    def out_copy(i, slot):
        return pltpu.make_async_copy(out_buf.at[slot], o_hbm.at[i],
                                     out_sem.at[slot])

    for k in range(_K):
        in_copy(k, k).start()

    def step(i, carry):
        slot = jax.lax.rem(i, _K)
        in_copy(i, slot).wait()

        @pl.when(i >= _K)
        def _():
            out_copy(i - _K, slot).wait()

        out_buf[slot] = in_buf[slot] * scale

        out_copy(i, slot).start()

        @pl.when(i + _K < n)
        def _():
            in_copy(i + _K, slot).start()

        return carry

    jax.lax.fori_loop(0, n, step, 0)

    for k in range(_K):
        out_copy(n - _K + k, k).wait()


def kernel(x, stage, embeddings):
    b, c, h, w = x.shape
    x3 = x.reshape(b, c, h * w)
    stage_arr = jnp.asarray(stage, jnp.int32).reshape((1,))
    emb_t = embeddings.T

    out = pl.pallas_call(
        _scale_body,
        in_specs=[
            pl.BlockSpec(memory_space=pltpu.SMEM),
            pl.BlockSpec(memory_space=pltpu.VMEM),
            pl.BlockSpec(memory_space=pltpu.HBM),
        ],
        out_specs=pl.BlockSpec(memory_space=pltpu.HBM),
        out_shape=jax.ShapeDtypeStruct((b, c, h * w), x.dtype),
        scratch_shapes=[
            pltpu.VMEM((_K, c, h * w), x.dtype),
            pltpu.VMEM((_K, c, h * w), x.dtype),
            pltpu.SemaphoreType.DMA((_K,)),
            pltpu.SemaphoreType.DMA((_K,)),
        ],
    )(stage_arr, emb_t, x3)
    return out.reshape(b, c, h, w)


# speedup vs baseline: 4.0172x; 3.7958x over previous
import jax
import jax.numpy as jnp
from jax.experimental import pallas as pl
from jax.experimental.pallas import tpu as pltpu

_B = 4


def _scale_kernel(stage_ref, emb_ref, x_ref, o_ref):
    s = stage_ref[0]
    emb = emb_ref[...]
    row = jax.lax.broadcasted_iota(jnp.int32, emb.shape, 0)
    scale = jnp.sum(jnp.where(row == s, emb, 0.0), axis=0)
    o_ref[...] = x_ref[...] * scale[None, None, :]


def kernel(x, stage, embeddings):
    b, c, h, w = x.shape
    xt = jnp.transpose(x, (0, 2, 3, 1)).reshape(b, h * w, c)
    stage_arr = jnp.asarray(stage, jnp.int32).reshape((1,))

    out = pl.pallas_call(
        _scale_kernel,
        grid=(b // _B,),
        in_specs=[
            pl.BlockSpec(memory_space=pltpu.SMEM),
            pl.BlockSpec(embeddings.shape, lambda i: (0, 0)),
            pl.BlockSpec((_B, h * w, c), lambda i: (i, 0, 0)),
        ],
        out_specs=pl.BlockSpec((_B, h * w, c), lambda i: (i, 0, 0)),
        out_shape=jax.ShapeDtypeStruct((b, h * w, c), x.dtype),
        compiler_params=pltpu.CompilerParams(
            dimension_semantics=("arbitrary",),
        ),
    )(stage_arr, embeddings, xt)
    return out.reshape(b, h, w, c).transpose(0, 3, 1, 2)
